# baseline (speedup 1.0000x reference)
import jax
import jax.numpy as jnp
from jax import lax
from jax.experimental import pallas as pl
from jax.experimental.pallas import tpu as pltpu

_N_BINS = 15
_ROWS_PER_BLOCK = 256


def _ece_body(bb_ref, x_ref, t_ref, o_ref, scr):
    i = pl.program_id(0)
    nsteps = pl.num_programs(0)

    @pl.when(i == 0)
    def _init():
        scr[...] = jnp.zeros_like(scr)

    r = x_ref.shape[0]
    c = x_ref.shape[1] // 4
    conf = jnp.ones((r,), dtype=jnp.float32)
    accrow = jnp.zeros((r,), dtype=jnp.float32)
    t = t_ref[...]
    for j in range(3):
        x = x_ref[:, pl.ds(j * c, c)]
        m = jnp.max(x, axis=1)
        s = jnp.sum(jnp.exp(x - m[:, None]), axis=1)
        iota = lax.broadcasted_iota(jnp.int32, x.shape, 1)
        idx = jnp.min(
            jnp.where(x == m[:, None], iota, jnp.int32(2**30)), axis=1
        )
        conf = conf * (1.0 / s)
        accrow = accrow + (idx == t[:, j + 1]).astype(jnp.float32)

    bb = bb_ref[...]
    cmp = (conf[:, None] > bb).astype(jnp.int32)
    binid = jnp.sum(cmp, axis=1) - 1
    onehot = (
        binid[:, None] == lax.broadcasted_iota(jnp.int32, (r, 16), 1)
    ).astype(jnp.float32)
    scr[0:1, :] += jnp.sum(onehot, axis=0)[None, :]
    scr[1:2, :] += jnp.sum(conf[:, None] * onehot, axis=0)[None, :]
    scr[2:3, :] += jnp.sum(accrow[:, None] * onehot, axis=0)[None, :]

    @pl.when(i == nsteps - 1)
    def _finish():
        counts = scr[0:1, :]
        csum = scr[1:2, :]
        asum = scr[2:3, :]
        n_total = jnp.float32(r) * jnp.float32(nsteps)
        safe = jnp.maximum(counts, 1.0)
        acc_in_bin = asum / (safe * 3.0)
        avg_conf_in_bin = csum / safe
        term = jnp.abs(avg_conf_in_bin - acc_in_bin) * (counts / n_total)
        o_ref[...] = jnp.sum(
            jnp.where(counts > 0, term, 0.0), axis=1, keepdims=True
        )


def kernel(logits, targets):
    n, p, c = logits.shape
    x2 = logits.reshape(n, p * c)
    t = targets.astype(jnp.int32)
    bb = jnp.linspace(0.0, 1.0, _N_BINS + 1).reshape(1, _N_BINS + 1)
    r = _ROWS_PER_BLOCK
    grid = n // r
    out = pl.pallas_call(
        _ece_body,
        grid=(grid,),
        in_specs=[
            pl.BlockSpec((1, _N_BINS + 1), lambda i: (0, 0)),
            pl.BlockSpec((r, p * c), lambda i: (i, 0)),
            pl.BlockSpec((r, p), lambda i: (i, 0)),
        ],
        out_specs=pl.BlockSpec((1, 1), lambda i: (0, 0)),
        out_shape=jax.ShapeDtypeStruct((1, 1), jnp.float32),
        scratch_shapes=[pltpu.VMEM((4, _N_BINS + 1), jnp.float32)],
    )(bb, x2, t)
    return out.reshape(1)
